# baseline (speedup 1.0000x reference)
import functools

import jax
import jax.numpy as jnp
from jax.experimental import pallas as pl
from jax.experimental.pallas import tpu as pltpu


def _sigmoid(v):
    return 0.5 * jnp.tanh(0.5 * v) + 0.5


def _tree_lstm_block(hs, nch, x_ref, nh_ref, nc_ref, m_ref,
                     wx_ref, bx_ref, wfg_ref, bfg_ref, waggr_ref, baggr_ref,
                     h_ref, c_ref):
    xw = jnp.dot(x_ref[...], wx_ref[...], preferred_element_type=jnp.float32)
    xw = (xw + bx_ref[...]) * m_ref[...]
    iou_input = xw[:, : 3 * hs]
    f_input = xw[:, 3 * hs:]

    nh = nh_ref[...]
    nh_ch = [nh[:, ch * hs:(ch + 1) * hs] for ch in range(nch)]
    h_sum = nh_ch[0]
    for ch in range(1, nch):
        h_sum = h_sum + nh_ch[ch]
    iou_aggr = jnp.dot(h_sum, waggr_ref[...],
                       preferred_element_type=jnp.float32) + baggr_ref[...]

    nc = nc_ref[...]
    fb = bfg_ref[...] + f_input
    c_aggr = None
    for ch in range(nch):
        fg = jnp.dot(nh_ch[ch], wfg_ref[...],
                     preferred_element_type=jnp.float32)
        f = _sigmoid(fg + fb)
        contrib = f * nc[:, ch * hs:(ch + 1) * hs]
        c_aggr = contrib if c_aggr is None else c_aggr + contrib

    iou = iou_input + iou_aggr
    i = _sigmoid(iou[:, :hs])
    o = _sigmoid(iou[:, hs: 2 * hs])
    u = jnp.tanh(iou[:, 2 * hs:])
    c = i * u + c_aggr
    h_ref[...] = o * jnp.tanh(c)
    c_ref[...] = c


def kernel(x, neighbour_h, neighbour_c, mask,
           W_iou, b_iou, W_fin, b_fin, W_fg, b_fg, W_aggr, b_aggr):
    n, xs = x.shape
    _, nch, hs = neighbour_h.shape

    bn = 1000
    if n % bn:
        bn = 8
    grid = (n // bn,)

    nh2 = neighbour_h.reshape(n, nch * hs)
    nc2 = neighbour_c.reshape(n, nch * hs)
    maskf = mask.astype(jnp.float32).reshape(n, 1)
    wx = jnp.concatenate([W_iou, W_fin], axis=1)
    bx = jnp.concatenate([b_iou, b_fin]).reshape(1, 4 * hs)
    bfg = b_fg.reshape(1, hs)
    baggr = b_aggr.reshape(1, 3 * hs)

    row = lambda i: (i, 0)
    rep2 = lambda i: (0, 0)

    h_out, c_out = pl.pallas_call(
        functools.partial(_tree_lstm_block, hs, nch),
        grid=grid,
        in_specs=[
            pl.BlockSpec((bn, xs), row),
            pl.BlockSpec((bn, nch * hs), row),
            pl.BlockSpec((bn, nch * hs), row),
            pl.BlockSpec((bn, 1), row),
            pl.BlockSpec((xs, 4 * hs), rep2),
            pl.BlockSpec((1, 4 * hs), rep2),
            pl.BlockSpec((hs, hs), rep2),
            pl.BlockSpec((1, hs), rep2),
            pl.BlockSpec((hs, 3 * hs), rep2),
            pl.BlockSpec((1, 3 * hs), rep2),
        ],
        out_specs=[
            pl.BlockSpec((bn, hs), row),
            pl.BlockSpec((bn, hs), row),
        ],
        out_shape=[
            jax.ShapeDtypeStruct((n, hs), jnp.float32),
            jax.ShapeDtypeStruct((n, hs), jnp.float32),
        ],
        compiler_params=pltpu.CompilerParams(
            dimension_semantics=("arbitrary",),
        ),
    )(x, nh2, nc2, maskf, wx, bx, W_fg, bfg, W_aggr, baggr)
    return h_out, c_out


# speedup vs baseline: 2.0460x; 2.0460x over previous
import functools

import jax
import jax.numpy as jnp
from jax.experimental import pallas as pl
from jax.experimental.pallas import tpu as pltpu


def _sigmoid(v):
    return 0.5 * jnp.tanh(0.5 * v) + 0.5


def _tree_lstm_block(nblk, bn, hs, nch,
                     x_ref, nh_hbm, nc_hbm, m_ref,
                     wx_ref, bx_ref, wfg_ref, bfg_ref, waggr_ref, baggr_ref,
                     h_ref, c_ref,
                     nh_buf, nc_buf, sem):
    i = pl.program_id(0)
    slot = jax.lax.rem(i, 2)
    nxt = jax.lax.rem(i + 1, 2)

    def start(blk, sl):
        base = blk * bn
        for ch in range(nch):
            pltpu.make_async_copy(
                nh_hbm.at[pl.ds(base, bn), ch],
                nh_buf.at[sl, ch],
                sem.at[sl, 0, ch]).start()
            pltpu.make_async_copy(
                nc_hbm.at[pl.ds(base, bn), ch],
                nc_buf.at[sl, ch],
                sem.at[sl, 1, ch]).start()

    @pl.when(i == 0)
    def _():
        start(i, slot)

    @pl.when(i + 1 < nblk)
    def _():
        start(i + 1, nxt)

    xw = jnp.dot(x_ref[...], wx_ref[...], preferred_element_type=jnp.float32)
    xw = (xw + bx_ref[...]) * m_ref[...]
    iou_input = xw[:, : 3 * hs]
    f_input = xw[:, 3 * hs:]

    for ch in range(nch):
        pltpu.make_async_copy(
            nh_hbm.at[pl.ds(0, bn), ch], nh_buf.at[slot, ch],
            sem.at[slot, 0, ch]).wait()
        pltpu.make_async_copy(
            nc_hbm.at[pl.ds(0, bn), ch], nc_buf.at[slot, ch],
            sem.at[slot, 1, ch]).wait()

    nh_ch = [nh_buf[slot, ch] for ch in range(nch)]
    h_sum = nh_ch[0]
    for ch in range(1, nch):
        h_sum = h_sum + nh_ch[ch]
    iou_aggr = jnp.dot(h_sum, waggr_ref[...],
                       preferred_element_type=jnp.float32) + baggr_ref[...]

    fb = bfg_ref[...] + f_input
    c_aggr = None
    for ch in range(nch):
        fg = jnp.dot(nh_ch[ch], wfg_ref[...],
                     preferred_element_type=jnp.float32)
        f = _sigmoid(fg + fb)
        contrib = f * nc_buf[slot, ch]
        c_aggr = contrib if c_aggr is None else c_aggr + contrib

    iou = iou_input + iou_aggr
    i_g = _sigmoid(iou[:, :hs])
    o_g = _sigmoid(iou[:, hs: 2 * hs])
    u_g = jnp.tanh(iou[:, 2 * hs:])
    c = i_g * u_g + c_aggr
    h_ref[...] = o_g * jnp.tanh(c)
    c_ref[...] = c


def kernel(x, neighbour_h, neighbour_c, mask,
           W_iou, b_iou, W_fin, b_fin, W_fg, b_fg, W_aggr, b_aggr):
    n, xs = x.shape
    _, nch, hs = neighbour_h.shape

    bn = 1000
    if n % bn:
        bn = 8
    nblk = n // bn
    grid = (nblk,)

    maskf = mask.astype(jnp.float32).reshape(n, 1)
    wx = jnp.concatenate([W_iou, W_fin], axis=1)
    bx = jnp.concatenate([b_iou, b_fin]).reshape(1, 4 * hs)
    bfg = b_fg.reshape(1, hs)
    baggr = b_aggr.reshape(1, 3 * hs)

    row = lambda i: (i, 0)
    rep2 = lambda i: (0, 0)

    h_out, c_out = pl.pallas_call(
        functools.partial(_tree_lstm_block, nblk, bn, hs, nch),
        grid=grid,
        in_specs=[
            pl.BlockSpec((bn, xs), row),
            pl.BlockSpec(memory_space=pl.ANY),
            pl.BlockSpec(memory_space=pl.ANY),
            pl.BlockSpec((bn, 1), row),
            pl.BlockSpec((xs, 4 * hs), rep2),
            pl.BlockSpec((1, 4 * hs), rep2),
            pl.BlockSpec((hs, hs), rep2),
            pl.BlockSpec((1, hs), rep2),
            pl.BlockSpec((hs, 3 * hs), rep2),
            pl.BlockSpec((1, 3 * hs), rep2),
        ],
        out_specs=[
            pl.BlockSpec((bn, hs), row),
            pl.BlockSpec((bn, hs), row),
        ],
        out_shape=[
            jax.ShapeDtypeStruct((n, hs), jnp.float32),
            jax.ShapeDtypeStruct((n, hs), jnp.float32),
        ],
        scratch_shapes=[
            pltpu.VMEM((2, nch, bn, hs), jnp.float32),
            pltpu.VMEM((2, nch, bn, hs), jnp.float32),
            pltpu.SemaphoreType.DMA((2, 2, nch)),
        ],
        compiler_params=pltpu.CompilerParams(
            dimension_semantics=("arbitrary",),
        ),
    )(x, neighbour_h, neighbour_c, maskf, wx, bx, W_fg, bfg, W_aggr, baggr)
    return h_out, c_out


# speedup vs baseline: 2.2333x; 1.0916x over previous
import functools

import jax
import jax.numpy as jnp
from jax.experimental import pallas as pl
from jax.experimental.pallas import tpu as pltpu


def _sigmoid(v):
    return 0.5 * jnp.tanh(0.5 * v) + 0.5


def _tree_lstm_block(nblk, bn, hs, nch,
                     x_ref, nh_hbm, nc_hbm, m_ref,
                     wx_ref, bx_ref, wfg_ref, bfg_ref, waggr_ref, baggr_ref,
                     h_ref, c_ref,
                     nh_buf, nc_buf, sem):
    i = pl.program_id(0)
    slot = jax.lax.rem(i, 2)
    nxt = jax.lax.rem(i + 1, 2)

    def start(blk, sl):
        base = blk * bn
        for ch in range(nch):
            pltpu.make_async_copy(
                nh_hbm.at[pl.ds(base, bn), ch],
                nh_buf.at[sl, ch],
                sem.at[sl, 0, ch]).start()
            pltpu.make_async_copy(
                nc_hbm.at[pl.ds(base, bn), ch],
                nc_buf.at[sl, ch],
                sem.at[sl, 1, ch]).start()

    @pl.when(i == 0)
    def _():
        start(i, slot)

    @pl.when(i + 1 < nblk)
    def _():
        start(i + 1, nxt)

    xw = jnp.dot(x_ref[...], wx_ref[...], preferred_element_type=jnp.float32)
    xw = (xw + bx_ref[...]) * m_ref[...]
    iou_input = xw[:, : 3 * hs]
    f_input = xw[:, 3 * hs:]

    for ch in range(nch):
        pltpu.make_async_copy(
            nh_hbm.at[pl.ds(0, bn), ch], nh_buf.at[slot, ch],
            sem.at[slot, 0, ch]).wait()
        pltpu.make_async_copy(
            nc_hbm.at[pl.ds(0, bn), ch], nc_buf.at[slot, ch],
            sem.at[slot, 1, ch]).wait()

    nh_ch = [nh_buf[slot, ch] for ch in range(nch)]
    h_sum = nh_ch[0]
    for ch in range(1, nch):
        h_sum = h_sum + nh_ch[ch]
    iou_aggr = jnp.dot(h_sum, waggr_ref[...],
                       preferred_element_type=jnp.float32) + baggr_ref[...]

    fb = bfg_ref[...] + f_input
    c_aggr = None
    for ch in range(nch):
        fg = jnp.dot(nh_ch[ch], wfg_ref[...],
                     preferred_element_type=jnp.float32)
        f = _sigmoid(fg + fb)
        contrib = f * nc_buf[slot, ch]
        c_aggr = contrib if c_aggr is None else c_aggr + contrib

    iou = iou_input + iou_aggr
    i_g = _sigmoid(iou[:, :hs])
    o_g = _sigmoid(iou[:, hs: 2 * hs])
    u_g = jnp.tanh(iou[:, 2 * hs:])
    c = i_g * u_g + c_aggr
    h_ref[...] = o_g * jnp.tanh(c)
    c_ref[...] = c


def kernel(x, neighbour_h, neighbour_c, mask,
           W_iou, b_iou, W_fin, b_fin, W_fg, b_fg, W_aggr, b_aggr):
    n, xs = x.shape
    _, nch, hs = neighbour_h.shape

    bn = 2000
    if n % bn:
        bn = 8
    nblk = n // bn
    grid = (nblk,)

    maskf = mask.astype(jnp.float32).reshape(n, 1)
    wx = jnp.concatenate([W_iou, W_fin], axis=1)
    bx = jnp.concatenate([b_iou, b_fin]).reshape(1, 4 * hs)
    bfg = b_fg.reshape(1, hs)
    baggr = b_aggr.reshape(1, 3 * hs)

    row = lambda i: (i, 0)
    rep2 = lambda i: (0, 0)

    h_out, c_out = pl.pallas_call(
        functools.partial(_tree_lstm_block, nblk, bn, hs, nch),
        grid=grid,
        in_specs=[
            pl.BlockSpec((bn, xs), row),
            pl.BlockSpec(memory_space=pl.ANY),
            pl.BlockSpec(memory_space=pl.ANY),
            pl.BlockSpec((bn, 1), row),
            pl.BlockSpec((xs, 4 * hs), rep2),
            pl.BlockSpec((1, 4 * hs), rep2),
            pl.BlockSpec((hs, hs), rep2),
            pl.BlockSpec((1, hs), rep2),
            pl.BlockSpec((hs, 3 * hs), rep2),
            pl.BlockSpec((1, 3 * hs), rep2),
        ],
        out_specs=[
            pl.BlockSpec((bn, hs), row),
            pl.BlockSpec((bn, hs), row),
        ],
        out_shape=[
            jax.ShapeDtypeStruct((n, hs), jnp.float32),
            jax.ShapeDtypeStruct((n, hs), jnp.float32),
        ],
        scratch_shapes=[
            pltpu.VMEM((2, nch, bn, hs), jnp.float32),
            pltpu.VMEM((2, nch, bn, hs), jnp.float32),
            pltpu.SemaphoreType.DMA((2, 2, nch)),
        ],
        compiler_params=pltpu.CompilerParams(
            dimension_semantics=("arbitrary",),
        ),
    )(x, neighbour_h, neighbour_c, maskf, wx, bx, W_fg, bfg, W_aggr, baggr)
    return h_out, c_out


# speedup vs baseline: 2.2529x; 1.0088x over previous
import functools

import jax
import jax.numpy as jnp
from jax.experimental import pallas as pl
from jax.experimental.pallas import tpu as pltpu


def _sigmoid(v):
    return 0.5 * jnp.tanh(0.5 * v) + 0.5


def _tree_lstm_block(nblk, bn, hs, nch,
                     x_ref, nh_hbm, nc_hbm, m_ref,
                     wx_ref, bx_ref, wfg_ref, bfg_ref, waggr_ref, baggr_ref,
                     h_ref, c_ref,
                     nh_buf, nc_buf, sem):
    i = pl.program_id(0)
    slot = jax.lax.rem(i, 2)
    nxt = jax.lax.rem(i + 1, 2)

    def start(blk, sl):
        base = blk * bn
        for ch in range(nch):
            pltpu.make_async_copy(
                nh_hbm.at[pl.ds(base, bn), ch],
                nh_buf.at[sl, ch],
                sem.at[sl, 0, ch]).start()
            pltpu.make_async_copy(
                nc_hbm.at[pl.ds(base, bn), ch],
                nc_buf.at[sl, ch],
                sem.at[sl, 1, ch]).start()

    @pl.when(i == 0)
    def _():
        start(i, slot)

    @pl.when(i + 1 < nblk)
    def _():
        start(i + 1, nxt)

    xw = jnp.dot(x_ref[...], wx_ref[...], preferred_element_type=jnp.float32)
    xw = (xw + bx_ref[...]) * m_ref[...]
    iou_input = xw[:, : 3 * hs]
    f_input = xw[:, 3 * hs:]

    for ch in range(nch):
        pltpu.make_async_copy(
            nh_hbm.at[pl.ds(0, bn), ch], nh_buf.at[slot, ch],
            sem.at[slot, 0, ch]).wait()
        pltpu.make_async_copy(
            nc_hbm.at[pl.ds(0, bn), ch], nc_buf.at[slot, ch],
            sem.at[slot, 1, ch]).wait()

    nh_ch = [nh_buf[slot, ch] for ch in range(nch)]
    h_sum = nh_ch[0]
    for ch in range(1, nch):
        h_sum = h_sum + nh_ch[ch]
    iou_aggr = jnp.dot(h_sum, waggr_ref[...],
                       preferred_element_type=jnp.float32) + baggr_ref[...]

    fb = bfg_ref[...] + f_input
    c_aggr = None
    for ch in range(nch):
        fg = jnp.dot(nh_ch[ch], wfg_ref[...],
                     preferred_element_type=jnp.float32)
        f = _sigmoid(fg + fb)
        contrib = f * nc_buf[slot, ch]
        c_aggr = contrib if c_aggr is None else c_aggr + contrib

    iou = iou_input + iou_aggr
    i_g = _sigmoid(iou[:, :hs])
    o_g = _sigmoid(iou[:, hs: 2 * hs])
    u_g = jnp.tanh(iou[:, 2 * hs:])
    c = i_g * u_g + c_aggr
    h_ref[...] = o_g * jnp.tanh(c)
    c_ref[...] = c


def kernel(x, neighbour_h, neighbour_c, mask,
           W_iou, b_iou, W_fin, b_fin, W_fg, b_fg, W_aggr, b_aggr):
    n, xs = x.shape
    _, nch, hs = neighbour_h.shape

    bn = 4000
    if n % bn:
        bn = 8
    nblk = n // bn
    grid = (nblk,)

    maskf = mask.astype(jnp.float32).reshape(n, 1)
    wx = jnp.concatenate([W_iou, W_fin], axis=1)
    bx = jnp.concatenate([b_iou, b_fin]).reshape(1, 4 * hs)
    bfg = b_fg.reshape(1, hs)
    baggr = b_aggr.reshape(1, 3 * hs)

    row = lambda i: (i, 0)
    rep2 = lambda i: (0, 0)

    h_out, c_out = pl.pallas_call(
        functools.partial(_tree_lstm_block, nblk, bn, hs, nch),
        grid=grid,
        in_specs=[
            pl.BlockSpec((bn, xs), row),
            pl.BlockSpec(memory_space=pl.ANY),
            pl.BlockSpec(memory_space=pl.ANY),
            pl.BlockSpec((bn, 1), row),
            pl.BlockSpec((xs, 4 * hs), rep2),
            pl.BlockSpec((1, 4 * hs), rep2),
            pl.BlockSpec((hs, hs), rep2),
            pl.BlockSpec((1, hs), rep2),
            pl.BlockSpec((hs, 3 * hs), rep2),
            pl.BlockSpec((1, 3 * hs), rep2),
        ],
        out_specs=[
            pl.BlockSpec((bn, hs), row),
            pl.BlockSpec((bn, hs), row),
        ],
        out_shape=[
            jax.ShapeDtypeStruct((n, hs), jnp.float32),
            jax.ShapeDtypeStruct((n, hs), jnp.float32),
        ],
        scratch_shapes=[
            pltpu.VMEM((2, nch, bn, hs), jnp.float32),
            pltpu.VMEM((2, nch, bn, hs), jnp.float32),
            pltpu.SemaphoreType.DMA((2, 2, nch)),
        ],
        compiler_params=pltpu.CompilerParams(
            dimension_semantics=("arbitrary",),
        ),
    )(x, neighbour_h, neighbour_c, maskf, wx, bx, W_fg, bfg, W_aggr, baggr)
    return h_out, c_out
